# Initial kernel scaffold; baseline (speedup 1.0000x reference)
import jax
import jax.numpy as jnp
from jax.experimental import pallas as pl


def kernel(x, edge_attr, edge_index, batch, params):
    raise NotImplementedError("write your pallas kernel here")


# speedup vs baseline: 1.1981x; 1.1981x over previous
import functools

import jax
import jax.numpy as jnp
from jax.experimental import pallas as pl
from jax.experimental.pallas import tpu as pltpu

N = 10000
E = 320000
F = 128
FE = 16
D = 256
OUT = 4

NP = 10240
NB_N = 10
BN = NP // NB_N
BE = 3200
NB_E = E // BE


def _ln_rows(y, g, b):
    mu = jnp.mean(y, axis=-1, keepdims=True)
    var = jnp.mean((y - mu) ** 2, axis=-1, keepdims=True)
    return (y - mu) * jax.lax.rsqrt(var + 1e-5) * g + b


def _node_enc_body(x_ref, w1_ref, b1_ref, w2_ref, b2_ref, g_ref, be_ref, o_ref):
    h1 = jnp.maximum(
        jnp.dot(x_ref[...], w1_ref[...], preferred_element_type=jnp.float32)
        + b1_ref[...], 0.0)
    y = jnp.dot(h1, w2_ref[...], preferred_element_type=jnp.float32) + b2_ref[...]
    o_ref[...] = _ln_rows(y, g_ref[...], be_ref[...])


def _node_encoder(xp, p):
    rep = lambda *_: (0, 0)
    return pl.pallas_call(
        _node_enc_body,
        grid=(NB_N,),
        in_specs=[
            pl.BlockSpec((BN, F), lambda i: (i, 0)),
            pl.BlockSpec((F, D), rep),
            pl.BlockSpec((1, D), rep),
            pl.BlockSpec((D, D), rep),
            pl.BlockSpec((1, D), rep),
            pl.BlockSpec((1, D), rep),
            pl.BlockSpec((1, D), rep),
        ],
        out_specs=pl.BlockSpec((BN, D), lambda i: (i, 0)),
        out_shape=jax.ShapeDtypeStruct((NP, D), jnp.float32),
    )(xp, p['ne_W1'], p['ne_b1'][None], p['ne_W2'], p['ne_b2'][None],
      p['ne_g'][None], p['ne_be'][None])


def _edge_enc_body(ea_ref, w1_ref, b1_ref, w2_ref, b2_ref, g_ref, be_ref,
                   v_ref, a0_ref, a1_ref):
    h1 = jnp.maximum(
        jnp.dot(ea_ref[...], w1_ref[...], preferred_element_type=jnp.float32)
        + b1_ref[...], 0.0)
    y = jnp.dot(h1, w2_ref[...], preferred_element_type=jnp.float32) + b2_ref[...]
    ln = _ln_rows(y, g_ref[...], be_ref[...])
    a0_ref[...] = jnp.sum(ln * v_ref[0:1, :], axis=-1, keepdims=True)
    a1_ref[...] = jnp.sum(ln * v_ref[1:2, :], axis=-1, keepdims=True)


def _edge_encoder(edge_attr, p, v01):
    rep = lambda *_: (0, 0)
    a0, a1 = pl.pallas_call(
        _edge_enc_body,
        grid=(NB_E,),
        in_specs=[
            pl.BlockSpec((BE, FE), lambda i: (i, 0)),
            pl.BlockSpec((FE, D), rep),
            pl.BlockSpec((1, D), rep),
            pl.BlockSpec((D, D), rep),
            pl.BlockSpec((1, D), rep),
            pl.BlockSpec((1, D), rep),
            pl.BlockSpec((1, D), rep),
            pl.BlockSpec((2, D), rep),
        ],
        out_specs=[
            pl.BlockSpec((BE, 1), lambda i: (i, 0)),
            pl.BlockSpec((BE, 1), lambda i: (i, 0)),
        ],
        out_shape=[
            jax.ShapeDtypeStruct((E, 1), jnp.float32),
            jax.ShapeDtypeStruct((E, 1), jnp.float32),
        ],
    )(edge_attr, p['ee_W1'], p['ee_b1'][None], p['ee_W2'], p['ee_b2'][None],
      p['ee_g'][None], p['ee_be'][None], v01)
    return a0.reshape(E), a1.reshape(E)


def _proj_body(h_ref, w_ref, att_ref, xs_ref, as_ref, ad_ref):
    xs = jnp.dot(h_ref[...], w_ref[...], preferred_element_type=jnp.float32)
    xs_ref[...] = xs
    as_ref[...] = jnp.sum(xs * att_ref[0:1, :], axis=-1, keepdims=True)
    ad_ref[...] = jnp.sum(xs * att_ref[1:2, :], axis=-1, keepdims=True)


def _proj(h, W, att_sd):
    rep = lambda *_: (0, 0)
    xs, a_s, a_d = pl.pallas_call(
        _proj_body,
        grid=(NB_N,),
        in_specs=[
            pl.BlockSpec((BN, D), lambda i: (i, 0)),
            pl.BlockSpec((D, D), rep),
            pl.BlockSpec((2, D), rep),
        ],
        out_specs=[
            pl.BlockSpec((BN, D), lambda i: (i, 0)),
            pl.BlockSpec((BN, 1), lambda i: (i, 0)),
            pl.BlockSpec((BN, 1), lambda i: (i, 0)),
        ],
        out_shape=[
            jax.ShapeDtypeStruct((NP, D), jnp.float32),
            jax.ShapeDtypeStruct((NP, 1), jnp.float32),
            jax.ShapeDtypeStruct((NP, 1), jnp.float32),
        ],
    )(h, W, att_sd)
    return xs, a_s.reshape(NP), a_d.reshape(NP)


def _assemble_body(acc_ref, xs_ref, cl_ref, bias_ref, g_ref, be_ref, o_ref):
    y = acc_ref[...] + cl_ref[...] * xs_ref[...] + bias_ref[...]
    o_ref[...] = _ln_rows(y, g_ref[...], be_ref[...])


def _assemble(acc, xs, coef_loop, bias, g, be):
    rep = lambda *_: (0, 0)
    return pl.pallas_call(
        _assemble_body,
        grid=(NB_N,),
        in_specs=[
            pl.BlockSpec((BN, D), lambda i: (i, 0)),
            pl.BlockSpec((BN, D), lambda i: (i, 0)),
            pl.BlockSpec((BN, 1), lambda i: (i, 0)),
            pl.BlockSpec((1, D), rep),
            pl.BlockSpec((1, D), rep),
            pl.BlockSpec((1, D), rep),
        ],
        out_specs=pl.BlockSpec((BN, D), lambda i: (i, 0)),
        out_shape=jax.ShapeDtypeStruct((NP, D), jnp.float32),
    )(acc, xs, coef_loop.reshape(NP, 1), bias[None], g[None], be[None])


def _dec_body(h_ref, w1_ref, b1_ref, w2_ref, b2_ref, o_ref):
    h1 = jnp.maximum(
        jnp.dot(h_ref[...], w1_ref[...], preferred_element_type=jnp.float32)
        + b1_ref[...], 0.0)
    o_ref[...] = (jnp.dot(h1, w2_ref[...], preferred_element_type=jnp.float32)
                  + b2_ref[...])


def _decoder(h, p):
    rep = lambda *_: (0, 0)
    w2p = jnp.zeros((D, 128), jnp.float32).at[:, :OUT].set(p['dec_W2'])
    b2p = jnp.zeros((128,), jnp.float32).at[:OUT].set(p['dec_b2'])
    out = pl.pallas_call(
        _dec_body,
        grid=(NB_N,),
        in_specs=[
            pl.BlockSpec((BN, D), lambda i: (i, 0)),
            pl.BlockSpec((D, D), rep),
            pl.BlockSpec((1, D), rep),
            pl.BlockSpec((D, 128), rep),
            pl.BlockSpec((1, 128), rep),
        ],
        out_specs=pl.BlockSpec((BN, 128), lambda i: (i, 0)),
        out_shape=jax.ShapeDtypeStruct((NP, 128), jnp.float32),
    )(h, p['dec_W1'], p['dec_b1'][None], w2p, b2p[None])
    return out[:N, :OUT]


def kernel(x, edge_attr, edge_index, batch, params):
    p = params
    src, dst = edge_index[0], edge_index[1]

    xp = jnp.zeros((NP, F), jnp.float32).at[:N].set(x)
    h = _node_encoder(xp, p)

    v01 = jnp.stack([p['gat0_We'] @ p['gat0_att_e'],
                     p['gat1_We'] @ p['gat1_att_e']], axis=0)
    ae0, ae1 = _edge_encoder(edge_attr, p, v01)
    ae_by_layer = (ae0, ae1)

    for l in range(2):
        att_sd = jnp.stack([p['gat%d_att_s' % l], p['gat%d_att_d' % l]], axis=0)
        xs, a_s, a_d = _proj(h, p['gat%d_W' % l], att_sd)
        ae = ae_by_layer[l]
        ae_loop = jnp.mean(ae)

        alpha = a_s[src] + a_d[dst] + ae
        alpha = jnp.where(alpha > 0, alpha, 0.2 * alpha)
        alpha_loop = a_s + a_d + ae_loop
        alpha_loop = jnp.where(alpha_loop > 0, alpha_loop, 0.2 * alpha_loop)
        m = jnp.maximum(jax.ops.segment_max(alpha, dst, num_segments=NP),
                        alpha_loop)
        ex = jnp.exp(alpha - m[dst])
        ex_loop = jnp.exp(alpha_loop - m)
        denom = jax.ops.segment_sum(ex, dst, num_segments=NP) + ex_loop
        inv = 1.0 / (denom + 1e-16)
        coef = ex * inv[dst]
        acc = jax.ops.segment_sum(xs[src] * coef[:, None], dst, num_segments=NP)

        h = _assemble(acc, xs, ex_loop * inv, p['gat%d_bias' % l],
                      p['gat%d_g' % l], p['gat%d_be' % l])

    return _decoder(h, p)


# speedup vs baseline: 4.0600x; 3.3886x over previous
import functools

import jax
import jax.numpy as jnp
from jax import lax
from jax.experimental import pallas as pl
from jax.experimental.pallas import tpu as pltpu
from jax.experimental.pallas import tpu_sc as plsc

N = 10000
E = 320000
F = 128
FE = 16
D = 256
OUT = 4

NP = 10240
NB_N = 10
BN = NP // NB_N
BE = 3200
NB_E = E // BE


def _ln_rows(y, g, b):
    mu = jnp.mean(y, axis=-1, keepdims=True)
    var = jnp.mean((y - mu) ** 2, axis=-1, keepdims=True)
    return (y - mu) * jax.lax.rsqrt(var + 1e-5) * g + b


def _node_enc_body(x_ref, w1_ref, b1_ref, w2_ref, b2_ref, g_ref, be_ref, o_ref):
    h1 = jnp.maximum(
        jnp.dot(x_ref[...], w1_ref[...], preferred_element_type=jnp.float32)
        + b1_ref[...], 0.0)
    y = jnp.dot(h1, w2_ref[...], preferred_element_type=jnp.float32) + b2_ref[...]
    o_ref[...] = _ln_rows(y, g_ref[...], be_ref[...])


def _node_encoder(xp, p):
    rep = lambda *_: (0, 0)
    return pl.pallas_call(
        _node_enc_body,
        grid=(NB_N,),
        in_specs=[
            pl.BlockSpec((BN, F), lambda i: (i, 0)),
            pl.BlockSpec((F, D), rep),
            pl.BlockSpec((1, D), rep),
            pl.BlockSpec((D, D), rep),
            pl.BlockSpec((1, D), rep),
            pl.BlockSpec((1, D), rep),
            pl.BlockSpec((1, D), rep),
        ],
        out_specs=pl.BlockSpec((BN, D), lambda i: (i, 0)),
        out_shape=jax.ShapeDtypeStruct((NP, D), jnp.float32),
    )(xp, p['ne_W1'], p['ne_b1'][None], p['ne_W2'], p['ne_b2'][None],
      p['ne_g'][None], p['ne_be'][None])


def _edge_enc_body(ea_ref, w1_ref, b1_ref, w2_ref, b2_ref, g_ref, be_ref,
                   v_ref, a0_ref, a1_ref):
    h1 = jnp.maximum(
        jnp.dot(ea_ref[...], w1_ref[...], preferred_element_type=jnp.float32)
        + b1_ref[...], 0.0)
    y = jnp.dot(h1, w2_ref[...], preferred_element_type=jnp.float32) + b2_ref[...]
    ln = _ln_rows(y, g_ref[...], be_ref[...])
    a0_ref[...] = jnp.sum(ln * v_ref[0:1, :], axis=-1, keepdims=True)
    a1_ref[...] = jnp.sum(ln * v_ref[1:2, :], axis=-1, keepdims=True)


def _edge_encoder(edge_attr, p, v01):
    rep = lambda *_: (0, 0)
    a0, a1 = pl.pallas_call(
        _edge_enc_body,
        grid=(NB_E,),
        in_specs=[
            pl.BlockSpec((BE, FE), lambda i: (i, 0)),
            pl.BlockSpec((FE, D), rep),
            pl.BlockSpec((1, D), rep),
            pl.BlockSpec((D, D), rep),
            pl.BlockSpec((1, D), rep),
            pl.BlockSpec((1, D), rep),
            pl.BlockSpec((1, D), rep),
            pl.BlockSpec((2, D), rep),
        ],
        out_specs=[
            pl.BlockSpec((BE, 1), lambda i: (i, 0)),
            pl.BlockSpec((BE, 1), lambda i: (i, 0)),
        ],
        out_shape=[
            jax.ShapeDtypeStruct((E, 1), jnp.float32),
            jax.ShapeDtypeStruct((E, 1), jnp.float32),
        ],
    )(edge_attr, p['ee_W1'], p['ee_b1'][None], p['ee_W2'], p['ee_b2'][None],
      p['ee_g'][None], p['ee_be'][None], v01)
    return a0.reshape(E), a1.reshape(E)


def _proj_body(h_ref, w_ref, att_ref, xs_ref, as_ref, ad_ref):
    xs = jnp.dot(h_ref[...], w_ref[...], preferred_element_type=jnp.float32)
    xs_ref[...] = xs
    as_ref[...] = jnp.sum(xs * att_ref[0:1, :], axis=-1, keepdims=True)
    ad_ref[...] = jnp.sum(xs * att_ref[1:2, :], axis=-1, keepdims=True)


def _proj(h, W, att_sd):
    rep = lambda *_: (0, 0)
    xs, a_s, a_d = pl.pallas_call(
        _proj_body,
        grid=(NB_N,),
        in_specs=[
            pl.BlockSpec((BN, D), lambda i: (i, 0)),
            pl.BlockSpec((D, D), rep),
            pl.BlockSpec((2, D), rep),
        ],
        out_specs=[
            pl.BlockSpec((BN, D), lambda i: (i, 0)),
            pl.BlockSpec((BN, 1), lambda i: (i, 0)),
            pl.BlockSpec((BN, 1), lambda i: (i, 0)),
        ],
        out_shape=[
            jax.ShapeDtypeStruct((NP, D), jnp.float32),
            jax.ShapeDtypeStruct((NP, 1), jnp.float32),
            jax.ShapeDtypeStruct((NP, 1), jnp.float32),
        ],
    )(h, W, att_sd)
    return xs, a_s.reshape(NP), a_d.reshape(NP)


def _assemble_body(acc_ref, xs_ref, cl_ref, bias_ref, g_ref, be_ref, o_ref):
    y = acc_ref[...] + cl_ref[...] * xs_ref[...] + bias_ref[...]
    o_ref[...] = _ln_rows(y, g_ref[...], be_ref[...])


def _assemble(acc, xs, coef_loop, bias, g, be):
    rep = lambda *_: (0, 0)
    return pl.pallas_call(
        _assemble_body,
        grid=(NB_N,),
        in_specs=[
            pl.BlockSpec((BN, D), lambda i: (i, 0)),
            pl.BlockSpec((BN, D), lambda i: (i, 0)),
            pl.BlockSpec((BN, 1), lambda i: (i, 0)),
            pl.BlockSpec((1, D), rep),
            pl.BlockSpec((1, D), rep),
            pl.BlockSpec((1, D), rep),
        ],
        out_specs=pl.BlockSpec((BN, D), lambda i: (i, 0)),
        out_shape=jax.ShapeDtypeStruct((NP, D), jnp.float32),
    )(acc, xs, coef_loop.reshape(NP, 1), bias[None], g[None], be[None])


def _dec_body(h_ref, w1_ref, b1_ref, w2_ref, b2_ref, o_ref):
    h1 = jnp.maximum(
        jnp.dot(h_ref[...], w1_ref[...], preferred_element_type=jnp.float32)
        + b1_ref[...], 0.0)
    o_ref[...] = (jnp.dot(h1, w2_ref[...], preferred_element_type=jnp.float32)
                  + b2_ref[...])


def _decoder(h, p):
    rep = lambda *_: (0, 0)
    w2p = jnp.zeros((D, 128), jnp.float32).at[:, :OUT].set(p['dec_W2'])
    b2p = jnp.zeros((128,), jnp.float32).at[:OUT].set(p['dec_b2'])
    out = pl.pallas_call(
        _dec_body,
        grid=(NB_N,),
        in_specs=[
            pl.BlockSpec((BN, D), lambda i: (i, 0)),
            pl.BlockSpec((D, D), rep),
            pl.BlockSpec((1, D), rep),
            pl.BlockSpec((D, 128), rep),
            pl.BlockSpec((1, 128), rep),
        ],
        out_specs=pl.BlockSpec((BN, 128), lambda i: (i, 0)),
        out_shape=jax.ShapeDtypeStruct((NP, 128), jnp.float32),
    )(h, p['dec_W1'], p['dec_b1'][None], w2p, b2p[None])
    return out[:N, :OUT]


_MESH = plsc.VectorSubcoreMesh(core_axis_name="c", subcore_axis_name="s")
_CHA = 128
_NCHA = E // _CHA
_CHA_FULL = _NCHA // 32
_CHA_LEFT = _NCHA - 32 * _CHA_FULL
_CHC = 64
_NCHC = E // _CHC
_CHC_FULL = _NCHC // 32
_CHC_LEFT = _NCHC - 32 * _CHC_FULL

_i32 = jnp.int32
_f32 = jnp.float32


def _worker_id():
    return lax.axis_index("s") * 2 + lax.axis_index("c")


@functools.partial(
    pl.kernel, mesh=_MESH,
    out_type=jax.ShapeDtypeStruct((E,), _f32),
    scratch_types=[pltpu.VMEM((_CHA,), _i32), pltpu.VMEM((_CHA,), _i32),
                   pltpu.VMEM((_CHA,), _f32), pltpu.VMEM((_CHA,), _f32),
                   pltpu.VMEM((_CHA,), _f32), pltpu.VMEM((_CHA,), _f32),
                   pltpu.SemaphoreType.DMA])
def _sc_alpha(src_hbm, dst_hbm, ae_hbm, as_hbm, ad_hbm, out_hbm,
              sbuf, dbuf, aebuf, asg, adg, albuf, sem):
    w = _worker_id()

    def chunk(cid):
        base = cid * _CHA
        pltpu.sync_copy(src_hbm.at[pl.ds(base, _CHA)], sbuf)
        pltpu.sync_copy(dst_hbm.at[pl.ds(base, _CHA)], dbuf)
        pltpu.sync_copy(ae_hbm.at[pl.ds(base, _CHA)], aebuf)
        pltpu.async_copy(as_hbm.at[sbuf], asg, sem).wait()
        pltpu.async_copy(ad_hbm.at[dbuf], adg, sem).wait()
        for g in range(_CHA // 16):
            sl = pl.ds(g * 16, 16)
            a = asg[sl] + adg[sl] + aebuf[sl]
            albuf[sl] = jnp.where(a > 0, a, 0.2 * a)
        pltpu.sync_copy(albuf, out_hbm.at[pl.ds(base, _CHA)])

    def body(i, _):
        chunk(w + i * 32)
        return 0

    lax.fori_loop(0, _CHA_FULL, body, 0)
    pl.when(w < _CHA_LEFT)(lambda: chunk(32 * _CHA_FULL + w))


@functools.partial(
    pl.kernel, mesh=_MESH,
    out_type=jax.ShapeDtypeStruct((E,), _f32),
    scratch_types=[pltpu.VMEM((_CHA,), _i32), pltpu.VMEM((_CHA,), _f32),
                   pltpu.VMEM((_CHA,), _f32), pltpu.VMEM((_CHA,), _f32),
                   pltpu.SemaphoreType.DMA])
def _sc_ex(dst_hbm, al_hbm, m_hbm, out_hbm, dbuf, albuf, mg, exbuf, sem):
    w = _worker_id()

    def chunk(cid):
        base = cid * _CHA
        pltpu.sync_copy(dst_hbm.at[pl.ds(base, _CHA)], dbuf)
        pltpu.sync_copy(al_hbm.at[pl.ds(base, _CHA)], albuf)
        pltpu.async_copy(m_hbm.at[dbuf], mg, sem).wait()
        for g in range(_CHA // 16):
            sl = pl.ds(g * 16, 16)
            exbuf[sl] = jnp.exp(albuf[sl] - mg[sl])
        pltpu.sync_copy(exbuf, out_hbm.at[pl.ds(base, _CHA)])

    def body(i, _):
        chunk(w + i * 32)
        return 0

    lax.fori_loop(0, _CHA_FULL, body, 0)
    pl.when(w < _CHA_LEFT)(lambda: chunk(32 * _CHA_FULL + w))


@functools.partial(
    pl.kernel, mesh=_MESH,
    out_type=jax.ShapeDtypeStruct((E, D), _f32),
    scratch_types=[pltpu.VMEM((_CHC,), _i32), pltpu.VMEM((_CHC,), _i32),
                   pltpu.VMEM((_CHC,), _f32), pltpu.VMEM((_CHC,), _f32),
                   pltpu.VMEM((_CHC,), _f32), pltpu.VMEM((_CHC, D), _f32),
                   pltpu.SemaphoreType.DMA, pltpu.SemaphoreType.DMA])
def _sc_msg(src_hbm, dst_hbm, ex_hbm, inv_hbm, xs_hbm, out_hbm,
            sbuf, dbuf, exbuf, invg, coefb, rows, sem, sem2):
    w = _worker_id()

    def chunk(cid):
        base = cid * _CHC
        pltpu.sync_copy(src_hbm.at[pl.ds(base, _CHC)], sbuf)
        pltpu.sync_copy(dst_hbm.at[pl.ds(base, _CHC)], dbuf)
        pltpu.sync_copy(ex_hbm.at[pl.ds(base, _CHC)], exbuf)
        gat = pltpu.async_copy(xs_hbm.at[sbuf], rows, sem2)
        pltpu.async_copy(inv_hbm.at[dbuf], invg, sem).wait()
        for g in range(_CHC // 16):
            sl = pl.ds(g * 16, 16)
            coefb[sl] = exbuf[sl] * invg[sl]
        gat.wait()
        for g in range(_CHC // 16):
            cg = coefb[pl.ds(g * 16, 16)]
            for r in range(16):
                row = g * 16 + r
                cs = cg[r]
                for q in range(D // 16):
                    qs = pl.ds(q * 16, 16)
                    rows[row, qs] = rows[row, qs] * cs
        pltpu.sync_copy(rows, out_hbm.at[pl.ds(base, _CHC)])

    def body(i, _):
        chunk(w + i * 32)
        return 0

    lax.fori_loop(0, _CHC_FULL, body, 0)
    pl.when(w < _CHC_LEFT)(lambda: chunk(32 * _CHC_FULL + w))


def kernel(x, edge_attr, edge_index, batch, params):
    p = params
    src, dst = edge_index[0], edge_index[1]

    xp = jnp.zeros((NP, F), jnp.float32).at[:N].set(x)
    h = _node_encoder(xp, p)

    v01 = jnp.stack([p['gat0_We'] @ p['gat0_att_e'],
                     p['gat1_We'] @ p['gat1_att_e']], axis=0)
    ae0, ae1 = _edge_encoder(edge_attr, p, v01)
    ae_by_layer = (ae0, ae1)

    for l in range(2):
        att_sd = jnp.stack([p['gat%d_att_s' % l], p['gat%d_att_d' % l]], axis=0)
        xs, a_s, a_d = _proj(h, p['gat%d_W' % l], att_sd)
        ae = ae_by_layer[l]
        ae_loop = jnp.mean(ae)

        alpha = _sc_alpha(src, dst, ae, a_s, a_d)
        alpha_loop = a_s + a_d + ae_loop
        alpha_loop = jnp.where(alpha_loop > 0, alpha_loop, 0.2 * alpha_loop)
        m = jnp.maximum(jax.ops.segment_max(alpha, dst, num_segments=NP),
                        alpha_loop)
        ex = _sc_ex(dst, alpha, m)
        ex_loop = jnp.exp(alpha_loop - m)
        denom = jax.ops.segment_sum(ex, dst, num_segments=NP) + ex_loop
        inv = 1.0 / (denom + 1e-16)
        msg = _sc_msg(src, dst, ex, inv, xs)
        acc = jax.ops.segment_sum(msg, dst, num_segments=NP)

        h = _assemble(acc, xs, ex_loop * inv, p['gat%d_bias' % l],
                      p['gat%d_g' % l], p['gat%d_be' % l])

    return _decoder(h, p)


# speedup vs baseline: 4.5632x; 1.1239x over previous
import functools

import jax
import jax.numpy as jnp
from jax import lax
from jax.experimental import pallas as pl
from jax.experimental.pallas import tpu as pltpu
from jax.experimental.pallas import tpu_sc as plsc

N = 10000
E = 320000
F = 128
FE = 16
D = 256
OUT = 4

NP = 10240
NB_N = 10
BN = NP // NB_N
BE = 3200
NB_E = E // BE


def _ln_rows(y, g, b):
    mu = jnp.mean(y, axis=-1, keepdims=True)
    var = jnp.mean((y - mu) ** 2, axis=-1, keepdims=True)
    return (y - mu) * jax.lax.rsqrt(var + 1e-5) * g + b


def _node_enc_body(x_ref, w1_ref, b1_ref, w2_ref, b2_ref, g_ref, be_ref, o_ref):
    h1 = jnp.maximum(
        jnp.dot(x_ref[...], w1_ref[...], preferred_element_type=jnp.float32)
        + b1_ref[...], 0.0)
    y = jnp.dot(h1, w2_ref[...], preferred_element_type=jnp.float32) + b2_ref[...]
    o_ref[...] = _ln_rows(y, g_ref[...], be_ref[...])


def _node_encoder(xp, p):
    rep = lambda *_: (0, 0)
    return pl.pallas_call(
        _node_enc_body,
        grid=(NB_N,),
        in_specs=[
            pl.BlockSpec((BN, F), lambda i: (i, 0)),
            pl.BlockSpec((F, D), rep),
            pl.BlockSpec((1, D), rep),
            pl.BlockSpec((D, D), rep),
            pl.BlockSpec((1, D), rep),
            pl.BlockSpec((1, D), rep),
            pl.BlockSpec((1, D), rep),
        ],
        out_specs=pl.BlockSpec((BN, D), lambda i: (i, 0)),
        out_shape=jax.ShapeDtypeStruct((NP, D), jnp.float32),
    )(xp, p['ne_W1'], p['ne_b1'][None], p['ne_W2'], p['ne_b2'][None],
      p['ne_g'][None], p['ne_be'][None])


def _edge_enc_body(ea_ref, w1_ref, b1_ref, w2_ref, b2_ref, g_ref, be_ref,
                   v_ref, a0_ref, a1_ref):
    h1 = jnp.maximum(
        jnp.dot(ea_ref[...], w1_ref[...], preferred_element_type=jnp.float32)
        + b1_ref[...], 0.0)
    y = jnp.dot(h1, w2_ref[...], preferred_element_type=jnp.float32) + b2_ref[...]
    ln = _ln_rows(y, g_ref[...], be_ref[...])
    a0_ref[...] = jnp.sum(ln * v_ref[0:1, :], axis=-1, keepdims=True)
    a1_ref[...] = jnp.sum(ln * v_ref[1:2, :], axis=-1, keepdims=True)


def _edge_encoder(edge_attr, p, v01):
    rep = lambda *_: (0, 0)
    a0, a1 = pl.pallas_call(
        _edge_enc_body,
        grid=(NB_E,),
        in_specs=[
            pl.BlockSpec((BE, FE), lambda i: (i, 0)),
            pl.BlockSpec((FE, D), rep),
            pl.BlockSpec((1, D), rep),
            pl.BlockSpec((D, D), rep),
            pl.BlockSpec((1, D), rep),
            pl.BlockSpec((1, D), rep),
            pl.BlockSpec((1, D), rep),
            pl.BlockSpec((2, D), rep),
        ],
        out_specs=[
            pl.BlockSpec((BE, 1), lambda i: (i, 0)),
            pl.BlockSpec((BE, 1), lambda i: (i, 0)),
        ],
        out_shape=[
            jax.ShapeDtypeStruct((E, 1), jnp.float32),
            jax.ShapeDtypeStruct((E, 1), jnp.float32),
        ],
    )(edge_attr, p['ee_W1'], p['ee_b1'][None], p['ee_W2'], p['ee_b2'][None],
      p['ee_g'][None], p['ee_be'][None], v01)
    return a0.reshape(E), a1.reshape(E)


def _proj_body(h_ref, w_ref, att_ref, xs_ref, as_ref, ad_ref):
    xs = jnp.dot(h_ref[...], w_ref[...], preferred_element_type=jnp.float32)
    xs_ref[...] = xs
    as_ref[...] = jnp.sum(xs * att_ref[0:1, :], axis=-1, keepdims=True)
    ad_ref[...] = jnp.sum(xs * att_ref[1:2, :], axis=-1, keepdims=True)


def _proj(h, W, att_sd):
    rep = lambda *_: (0, 0)
    xs, a_s, a_d = pl.pallas_call(
        _proj_body,
        grid=(NB_N,),
        in_specs=[
            pl.BlockSpec((BN, D), lambda i: (i, 0)),
            pl.BlockSpec((D, D), rep),
            pl.BlockSpec((2, D), rep),
        ],
        out_specs=[
            pl.BlockSpec((BN, D), lambda i: (i, 0)),
            pl.BlockSpec((BN, 1), lambda i: (i, 0)),
            pl.BlockSpec((BN, 1), lambda i: (i, 0)),
        ],
        out_shape=[
            jax.ShapeDtypeStruct((NP, D), jnp.float32),
            jax.ShapeDtypeStruct((NP, 1), jnp.float32),
            jax.ShapeDtypeStruct((NP, 1), jnp.float32),
        ],
    )(h, W, att_sd)
    return xs, a_s.reshape(NP), a_d.reshape(NP)


def _assemble_body(acc_ref, xs_ref, cl_ref, bias_ref, g_ref, be_ref, o_ref):
    y = acc_ref[...] + cl_ref[...] * xs_ref[...] + bias_ref[...]
    o_ref[...] = _ln_rows(y, g_ref[...], be_ref[...])


def _assemble(acc, xs, coef_loop, bias, g, be):
    rep = lambda *_: (0, 0)
    return pl.pallas_call(
        _assemble_body,
        grid=(NB_N,),
        in_specs=[
            pl.BlockSpec((BN, D), lambda i: (i, 0)),
            pl.BlockSpec((BN, D), lambda i: (i, 0)),
            pl.BlockSpec((BN, 1), lambda i: (i, 0)),
            pl.BlockSpec((1, D), rep),
            pl.BlockSpec((1, D), rep),
            pl.BlockSpec((1, D), rep),
        ],
        out_specs=pl.BlockSpec((BN, D), lambda i: (i, 0)),
        out_shape=jax.ShapeDtypeStruct((NP, D), jnp.float32),
    )(acc, xs, coef_loop.reshape(NP, 1), bias[None], g[None], be[None])


def _dec_body(h_ref, w1_ref, b1_ref, w2_ref, b2_ref, o_ref):
    h1 = jnp.maximum(
        jnp.dot(h_ref[...], w1_ref[...], preferred_element_type=jnp.float32)
        + b1_ref[...], 0.0)
    o_ref[...] = (jnp.dot(h1, w2_ref[...], preferred_element_type=jnp.float32)
                  + b2_ref[...])


def _decoder(h, p):
    rep = lambda *_: (0, 0)
    w2p = jnp.zeros((D, 128), jnp.float32).at[:, :OUT].set(p['dec_W2'])
    b2p = jnp.zeros((128,), jnp.float32).at[:OUT].set(p['dec_b2'])
    out = pl.pallas_call(
        _dec_body,
        grid=(NB_N,),
        in_specs=[
            pl.BlockSpec((BN, D), lambda i: (i, 0)),
            pl.BlockSpec((D, D), rep),
            pl.BlockSpec((1, D), rep),
            pl.BlockSpec((D, 128), rep),
            pl.BlockSpec((1, 128), rep),
        ],
        out_specs=pl.BlockSpec((BN, 128), lambda i: (i, 0)),
        out_shape=jax.ShapeDtypeStruct((NP, 128), jnp.float32),
    )(h, p['dec_W1'], p['dec_b1'][None], w2p, b2p[None])
    return out[:N, :OUT]


_MESH = plsc.VectorSubcoreMesh(core_axis_name="c", subcore_axis_name="s")
_CHA = 128
_NCHA = E // _CHA
_CHA_FULL = _NCHA // 32
_CHA_LEFT = _NCHA - 32 * _CHA_FULL
_CHC = 64
_NCHC = E // _CHC
_CHC_FULL = _NCHC // 32
_CHC_LEFT = _NCHC - 32 * _CHC_FULL

_i32 = jnp.int32
_f32 = jnp.float32


def _worker_id():
    return lax.axis_index("s") * 2 + lax.axis_index("c")


@functools.partial(
    pl.kernel, mesh=_MESH,
    out_type=jax.ShapeDtypeStruct((E,), _f32),
    scratch_types=[pltpu.VMEM((_CHA,), _i32), pltpu.VMEM((_CHA,), _i32),
                   pltpu.VMEM((_CHA,), _f32), pltpu.VMEM((_CHA,), _f32),
                   pltpu.VMEM((_CHA,), _f32), pltpu.VMEM((_CHA,), _f32),
                   pltpu.SemaphoreType.DMA])
def _sc_alpha(src_hbm, dst_hbm, ae_hbm, as_hbm, ad_hbm, out_hbm,
              sbuf, dbuf, aebuf, asg, adg, albuf, sem):
    w = _worker_id()

    def chunk(cid):
        base = cid * _CHA
        pltpu.sync_copy(src_hbm.at[pl.ds(base, _CHA)], sbuf)
        pltpu.sync_copy(dst_hbm.at[pl.ds(base, _CHA)], dbuf)
        pltpu.sync_copy(ae_hbm.at[pl.ds(base, _CHA)], aebuf)
        pltpu.async_copy(as_hbm.at[sbuf], asg, sem).wait()
        pltpu.async_copy(ad_hbm.at[dbuf], adg, sem).wait()
        for g in range(_CHA // 16):
            sl = pl.ds(g * 16, 16)
            a = asg[sl] + adg[sl] + aebuf[sl]
            albuf[sl] = jnp.where(a > 0, a, 0.2 * a)
        pltpu.sync_copy(albuf, out_hbm.at[pl.ds(base, _CHA)])

    def body(i, _):
        chunk(w + i * 32)
        return 0

    lax.fori_loop(0, _CHA_FULL, body, 0)
    pl.when(w < _CHA_LEFT)(lambda: chunk(32 * _CHA_FULL + w))


@functools.partial(
    pl.kernel, mesh=_MESH,
    out_type=jax.ShapeDtypeStruct((E,), _f32),
    scratch_types=[pltpu.VMEM((_CHA,), _i32), pltpu.VMEM((_CHA,), _f32),
                   pltpu.VMEM((_CHA,), _f32), pltpu.VMEM((_CHA,), _f32),
                   pltpu.SemaphoreType.DMA])
def _sc_ex(dst_hbm, al_hbm, m_hbm, out_hbm, dbuf, albuf, mg, exbuf, sem):
    w = _worker_id()

    def chunk(cid):
        base = cid * _CHA
        pltpu.sync_copy(dst_hbm.at[pl.ds(base, _CHA)], dbuf)
        pltpu.sync_copy(al_hbm.at[pl.ds(base, _CHA)], albuf)
        pltpu.async_copy(m_hbm.at[dbuf], mg, sem).wait()
        for g in range(_CHA // 16):
            sl = pl.ds(g * 16, 16)
            exbuf[sl] = jnp.exp(jnp.minimum(albuf[sl] - mg[sl], 80.0))
        pltpu.sync_copy(exbuf, out_hbm.at[pl.ds(base, _CHA)])

    def body(i, _):
        chunk(w + i * 32)
        return 0

    lax.fori_loop(0, _CHA_FULL, body, 0)
    pl.when(w < _CHA_LEFT)(lambda: chunk(32 * _CHA_FULL + w))


@functools.partial(
    pl.kernel, mesh=_MESH,
    out_type=jax.ShapeDtypeStruct((E, D), _f32),
    scratch_types=[pltpu.VMEM((_CHC,), _i32), pltpu.VMEM((_CHC,), _i32),
                   pltpu.VMEM((_CHC,), _f32), pltpu.VMEM((_CHC,), _f32),
                   pltpu.VMEM((_CHC,), _f32), pltpu.VMEM((_CHC, D), _f32),
                   pltpu.SemaphoreType.DMA, pltpu.SemaphoreType.DMA])
def _sc_msg(src_hbm, dst_hbm, ex_hbm, inv_hbm, xs_hbm, out_hbm,
            sbuf, dbuf, exbuf, invg, coefb, rows, sem, sem2):
    w = _worker_id()

    def chunk(cid):
        base = cid * _CHC
        pltpu.sync_copy(src_hbm.at[pl.ds(base, _CHC)], sbuf)
        pltpu.sync_copy(dst_hbm.at[pl.ds(base, _CHC)], dbuf)
        pltpu.sync_copy(ex_hbm.at[pl.ds(base, _CHC)], exbuf)
        gat = pltpu.async_copy(xs_hbm.at[sbuf], rows, sem2)
        pltpu.async_copy(inv_hbm.at[dbuf], invg, sem).wait()
        for g in range(_CHC // 16):
            sl = pl.ds(g * 16, 16)
            coefb[sl] = exbuf[sl] * invg[sl]
        gat.wait()
        for g in range(_CHC // 16):
            cg = coefb[pl.ds(g * 16, 16)]
            for r in range(16):
                row = g * 16 + r
                cs = cg[r]
                for q in range(D // 16):
                    qs = pl.ds(q * 16, 16)
                    rows[row, qs] = rows[row, qs] * cs
        pltpu.sync_copy(rows, out_hbm.at[pl.ds(base, _CHC)])

    def body(i, _):
        chunk(w + i * 32)
        return 0

    lax.fori_loop(0, _CHC_FULL, body, 0)
    pl.when(w < _CHC_LEFT)(lambda: chunk(32 * _CHC_FULL + w))


def kernel(x, edge_attr, edge_index, batch, params):
    p = params
    src, dst = edge_index[0], edge_index[1]

    xp = jnp.zeros((NP, F), jnp.float32).at[:N].set(x)
    h = _node_encoder(xp, p)

    v01 = jnp.stack([p['gat0_We'] @ p['gat0_att_e'],
                     p['gat1_We'] @ p['gat1_att_e']], axis=0)
    ae0, ae1 = _edge_encoder(edge_attr, p, v01)
    ae_by_layer = (ae0, ae1)

    for l in range(2):
        att_sd = jnp.stack([p['gat%d_att_s' % l], p['gat%d_att_d' % l]], axis=0)
        xs, a_s, a_d = _proj(h, p['gat%d_W' % l], att_sd)
        ae = ae_by_layer[l]
        ae_loop = jnp.mean(ae)

        alpha = _sc_alpha(src, dst, ae, a_s, a_d)
        alpha_loop = a_s + a_d + ae_loop
        alpha_loop = jnp.where(alpha_loop > 0, alpha_loop, 0.2 * alpha_loop)
        ex = _sc_ex(dst, alpha, alpha_loop)
        denom = jax.ops.segment_sum(ex, dst, num_segments=NP) + 1.0
        inv = 1.0 / (denom + 1e-16)
        msg = _sc_msg(src, dst, ex, inv, xs)
        acc = jax.ops.segment_sum(msg, dst, num_segments=NP)

        h = _assemble(acc, xs, inv, p['gat%d_bias' % l],
                      p['gat%d_g' % l], p['gat%d_be' % l])

    return _decoder(h, p)


# speedup vs baseline: 4.8882x; 1.0712x over previous
import functools

import jax
import jax.numpy as jnp
from jax import lax
from jax.experimental import pallas as pl
from jax.experimental.pallas import tpu as pltpu
from jax.experimental.pallas import tpu_sc as plsc

N = 10000
E = 320000
F = 128
FE = 16
D = 256
OUT = 4

NP = 10240
NB_N = 10
BN = NP // NB_N
BE = 3200
NB_E = E // BE


def _ln_rows(y, g, b):
    mu = jnp.mean(y, axis=-1, keepdims=True)
    var = jnp.mean((y - mu) ** 2, axis=-1, keepdims=True)
    return (y - mu) * jax.lax.rsqrt(var + 1e-5) * g + b


def _node_enc_body(x_ref, w1_ref, b1_ref, w2_ref, b2_ref, g_ref, be_ref, o_ref):
    h1 = jnp.maximum(
        jnp.dot(x_ref[...], w1_ref[...], preferred_element_type=jnp.float32)
        + b1_ref[...], 0.0)
    y = jnp.dot(h1, w2_ref[...], preferred_element_type=jnp.float32) + b2_ref[...]
    o_ref[...] = _ln_rows(y, g_ref[...], be_ref[...])


def _node_encoder(xp, p):
    rep = lambda *_: (0, 0)
    return pl.pallas_call(
        _node_enc_body,
        grid=(NB_N,),
        in_specs=[
            pl.BlockSpec((BN, F), lambda i: (i, 0)),
            pl.BlockSpec((F, D), rep),
            pl.BlockSpec((1, D), rep),
            pl.BlockSpec((D, D), rep),
            pl.BlockSpec((1, D), rep),
            pl.BlockSpec((1, D), rep),
            pl.BlockSpec((1, D), rep),
        ],
        out_specs=pl.BlockSpec((BN, D), lambda i: (i, 0)),
        out_shape=jax.ShapeDtypeStruct((NP, D), jnp.float32),
    )(xp, p['ne_W1'], p['ne_b1'][None], p['ne_W2'], p['ne_b2'][None],
      p['ne_g'][None], p['ne_be'][None])


def _edge_enc_body(ea_ref, w1_ref, b1_ref, w2_ref, b2_ref, g_ref, be_ref,
                   v_ref, a0_ref, a1_ref):
    h1 = jnp.maximum(
        jnp.dot(ea_ref[...], w1_ref[...], preferred_element_type=jnp.float32)
        + b1_ref[...], 0.0)
    y = jnp.dot(h1, w2_ref[...], preferred_element_type=jnp.float32) + b2_ref[...]
    ln = _ln_rows(y, g_ref[...], be_ref[...])
    a0_ref[...] = jnp.sum(ln * v_ref[0:1, :], axis=-1, keepdims=True)
    a1_ref[...] = jnp.sum(ln * v_ref[1:2, :], axis=-1, keepdims=True)


def _edge_encoder(edge_attr, p, v01):
    rep = lambda *_: (0, 0)
    a0, a1 = pl.pallas_call(
        _edge_enc_body,
        grid=(NB_E,),
        in_specs=[
            pl.BlockSpec((BE, FE), lambda i: (i, 0)),
            pl.BlockSpec((FE, D), rep),
            pl.BlockSpec((1, D), rep),
            pl.BlockSpec((D, D), rep),
            pl.BlockSpec((1, D), rep),
            pl.BlockSpec((1, D), rep),
            pl.BlockSpec((1, D), rep),
            pl.BlockSpec((2, D), rep),
        ],
        out_specs=[
            pl.BlockSpec((BE, 1), lambda i: (i, 0)),
            pl.BlockSpec((BE, 1), lambda i: (i, 0)),
        ],
        out_shape=[
            jax.ShapeDtypeStruct((E, 1), jnp.float32),
            jax.ShapeDtypeStruct((E, 1), jnp.float32),
        ],
    )(edge_attr, p['ee_W1'], p['ee_b1'][None], p['ee_W2'], p['ee_b2'][None],
      p['ee_g'][None], p['ee_be'][None], v01)
    return a0.reshape(E), a1.reshape(E)


def _proj_body(h_ref, w_ref, att_ref, xs_ref, as_ref, ad_ref):
    xs = jnp.dot(h_ref[...], w_ref[...], preferred_element_type=jnp.float32)
    xs_ref[...] = xs
    as_ref[...] = jnp.sum(xs * att_ref[0:1, :], axis=-1, keepdims=True)
    ad_ref[...] = jnp.sum(xs * att_ref[1:2, :], axis=-1, keepdims=True)


def _proj(h, W, att_sd):
    rep = lambda *_: (0, 0)
    xs, a_s, a_d = pl.pallas_call(
        _proj_body,
        grid=(NB_N,),
        in_specs=[
            pl.BlockSpec((BN, D), lambda i: (i, 0)),
            pl.BlockSpec((D, D), rep),
            pl.BlockSpec((2, D), rep),
        ],
        out_specs=[
            pl.BlockSpec((BN, D), lambda i: (i, 0)),
            pl.BlockSpec((BN, 1), lambda i: (i, 0)),
            pl.BlockSpec((BN, 1), lambda i: (i, 0)),
        ],
        out_shape=[
            jax.ShapeDtypeStruct((NP, D), jnp.float32),
            jax.ShapeDtypeStruct((NP, 1), jnp.float32),
            jax.ShapeDtypeStruct((NP, 1), jnp.float32),
        ],
    )(h, W, att_sd)
    return xs, a_s.reshape(NP), a_d.reshape(NP)


def _assemble_body(acc_ref, xs_ref, cl_ref, bias_ref, g_ref, be_ref, o_ref):
    y = acc_ref[...] + cl_ref[...] * xs_ref[...] + bias_ref[...]
    o_ref[...] = _ln_rows(y, g_ref[...], be_ref[...])


def _assemble(acc, xs, coef_loop, bias, g, be):
    rep = lambda *_: (0, 0)
    return pl.pallas_call(
        _assemble_body,
        grid=(NB_N,),
        in_specs=[
            pl.BlockSpec((BN, D), lambda i: (i, 0)),
            pl.BlockSpec((BN, D), lambda i: (i, 0)),
            pl.BlockSpec((BN, 1), lambda i: (i, 0)),
            pl.BlockSpec((1, D), rep),
            pl.BlockSpec((1, D), rep),
            pl.BlockSpec((1, D), rep),
        ],
        out_specs=pl.BlockSpec((BN, D), lambda i: (i, 0)),
        out_shape=jax.ShapeDtypeStruct((NP, D), jnp.float32),
    )(acc, xs, coef_loop.reshape(NP, 1), bias[None], g[None], be[None])


def _dec_body(h_ref, w1_ref, b1_ref, w2_ref, b2_ref, o_ref):
    h1 = jnp.maximum(
        jnp.dot(h_ref[...], w1_ref[...], preferred_element_type=jnp.float32)
        + b1_ref[...], 0.0)
    o_ref[...] = (jnp.dot(h1, w2_ref[...], preferred_element_type=jnp.float32)
                  + b2_ref[...])


def _decoder(h, p):
    rep = lambda *_: (0, 0)
    w2p = jnp.zeros((D, 128), jnp.float32).at[:, :OUT].set(p['dec_W2'])
    b2p = jnp.zeros((128,), jnp.float32).at[:OUT].set(p['dec_b2'])
    out = pl.pallas_call(
        _dec_body,
        grid=(NB_N,),
        in_specs=[
            pl.BlockSpec((BN, D), lambda i: (i, 0)),
            pl.BlockSpec((D, D), rep),
            pl.BlockSpec((1, D), rep),
            pl.BlockSpec((D, 128), rep),
            pl.BlockSpec((1, 128), rep),
        ],
        out_specs=pl.BlockSpec((BN, 128), lambda i: (i, 0)),
        out_shape=jax.ShapeDtypeStruct((NP, 128), jnp.float32),
    )(h, p['dec_W1'], p['dec_b1'][None], w2p, b2p[None])
    return out[:N, :OUT]


_MESH = plsc.VectorSubcoreMesh(core_axis_name="c", subcore_axis_name="s")
_CHA = 128
_NCHA = E // _CHA
_CHA_FULL = _NCHA // 32
_CHA_LEFT = _NCHA - 32 * _CHA_FULL
_CHC = 64
_NCHC = E // _CHC
_CHC_FULL = _NCHC // 32
_CHC_LEFT = _NCHC - 32 * _CHC_FULL

_i32 = jnp.int32
_f32 = jnp.float32


def _worker_id():
    return lax.axis_index("s") * 2 + lax.axis_index("c")


@functools.partial(
    pl.kernel, mesh=_MESH,
    out_type=jax.ShapeDtypeStruct((E,), _f32),
    scratch_types=[pltpu.VMEM((_CHA,), _i32), pltpu.VMEM((_CHA,), _i32),
                   pltpu.VMEM((_CHA,), _f32), pltpu.VMEM((_CHA,), _f32),
                   pltpu.VMEM((_CHA,), _f32), pltpu.VMEM((_CHA,), _f32),
                   pltpu.VMEM((_CHA,), _f32),
                   pltpu.SemaphoreType.DMA, pltpu.SemaphoreType.DMA])
def _sc_edge(src_hbm, dst_hbm, ae_hbm, as_hbm, ad_hbm, c_hbm, out_hbm,
             sbuf, dbuf, aebuf, asg, adg, cg, exbuf, sem, sem2):
    w = _worker_id()

    def chunk(cid):
        base = cid * _CHA
        pltpu.sync_copy(src_hbm.at[pl.ds(base, _CHA)], sbuf)
        pltpu.sync_copy(dst_hbm.at[pl.ds(base, _CHA)], dbuf)
        pltpu.sync_copy(ae_hbm.at[pl.ds(base, _CHA)], aebuf)
        g1 = pltpu.async_copy(as_hbm.at[sbuf], asg, sem2)
        g2 = pltpu.async_copy(ad_hbm.at[dbuf], adg, sem)
        g3 = pltpu.async_copy(c_hbm.at[dbuf], cg, sem)
        g1.wait()
        g2.wait()
        g3.wait()
        for g in range(_CHA // 16):
            sl = pl.ds(g * 16, 16)
            a = asg[sl] + adg[sl] + aebuf[sl]
            a = jnp.where(a > 0, a, 0.2 * a)
            exbuf[sl] = jnp.exp(jnp.minimum(a - cg[sl], 80.0))
        pltpu.sync_copy(exbuf, out_hbm.at[pl.ds(base, _CHA)])

    def body(i, _):
        chunk(w + i * 32)
        return 0

    lax.fori_loop(0, _CHA_FULL, body, 0)
    pl.when(w < _CHA_LEFT)(lambda: chunk(32 * _CHA_FULL + w))


@functools.partial(
    pl.kernel, mesh=_MESH,
    out_type=jax.ShapeDtypeStruct((E, D), _f32),
    scratch_types=[pltpu.VMEM((_CHC,), _i32), pltpu.VMEM((_CHC,), _i32),
                   pltpu.VMEM((_CHC,), _f32), pltpu.VMEM((_CHC,), _f32),
                   pltpu.VMEM((_CHC,), _f32), pltpu.VMEM((_CHC, D), _f32),
                   pltpu.SemaphoreType.DMA, pltpu.SemaphoreType.DMA])
def _sc_msg(src_hbm, dst_hbm, ex_hbm, inv_hbm, xs_hbm, out_hbm,
            sbuf, dbuf, exbuf, invg, coefb, rows, sem, sem2):
    w = _worker_id()

    def chunk(cid):
        base = cid * _CHC
        pltpu.sync_copy(src_hbm.at[pl.ds(base, _CHC)], sbuf)
        pltpu.sync_copy(dst_hbm.at[pl.ds(base, _CHC)], dbuf)
        pltpu.sync_copy(ex_hbm.at[pl.ds(base, _CHC)], exbuf)
        gat = pltpu.async_copy(xs_hbm.at[sbuf], rows, sem2)
        pltpu.async_copy(inv_hbm.at[dbuf], invg, sem).wait()
        for g in range(_CHC // 16):
            sl = pl.ds(g * 16, 16)
            coefb[sl] = exbuf[sl] * invg[sl]
        gat.wait()
        for g in range(_CHC // 16):
            cg = coefb[pl.ds(g * 16, 16)]
            for r in range(16):
                row = g * 16 + r
                cs = cg[r]
                for q in range(D // 16):
                    qs = pl.ds(q * 16, 16)
                    rows[row, qs] = rows[row, qs] * cs
        pltpu.sync_copy(rows, out_hbm.at[pl.ds(base, _CHC)])

    def body(i, _):
        chunk(w + i * 32)
        return 0

    lax.fori_loop(0, _CHC_FULL, body, 0)
    pl.when(w < _CHC_LEFT)(lambda: chunk(32 * _CHC_FULL + w))


def kernel(x, edge_attr, edge_index, batch, params):
    p = params
    src, dst = edge_index[0], edge_index[1]

    xp = jnp.zeros((NP, F), jnp.float32).at[:N].set(x)
    h = _node_encoder(xp, p)

    v01 = jnp.stack([p['gat0_We'] @ p['gat0_att_e'],
                     p['gat1_We'] @ p['gat1_att_e']], axis=0)
    ae0, ae1 = _edge_encoder(edge_attr, p, v01)
    ae_by_layer = (ae0, ae1)

    for l in range(2):
        att_sd = jnp.stack([p['gat%d_att_s' % l], p['gat%d_att_d' % l]], axis=0)
        xs, a_s, a_d = _proj(h, p['gat%d_W' % l], att_sd)
        ae = ae_by_layer[l]
        ae_loop = jnp.mean(ae)

        alpha_loop = a_s + a_d + ae_loop
        alpha_loop = jnp.where(alpha_loop > 0, alpha_loop, 0.2 * alpha_loop)
        ex = _sc_edge(src, dst, ae, a_s, a_d, alpha_loop)
        denom = jax.ops.segment_sum(ex, dst, num_segments=NP) + 1.0
        inv = 1.0 / (denom + 1e-16)
        msg = _sc_msg(src, dst, ex, inv, xs)
        acc = jax.ops.segment_sum(msg, dst, num_segments=NP)

        h = _assemble(acc, xs, inv, p['gat%d_bias' % l],
                      p['gat%d_g' % l], p['gat%d_be' % l])

    return _decoder(h, p)


# speedup vs baseline: 4.9439x; 1.0114x over previous
import functools

import jax
import jax.numpy as jnp
from jax import lax
from jax.experimental import pallas as pl
from jax.experimental.pallas import tpu as pltpu
from jax.experimental.pallas import tpu_sc as plsc

N = 10000
E = 320000
F = 128
FE = 16
D = 256
OUT = 4

NP = 10240
NB_N = 10
BN = NP // NB_N
BE = 3200
NB_E = E // BE


def _ln_rows(y, g, b):
    mu = jnp.mean(y, axis=-1, keepdims=True)
    var = jnp.mean((y - mu) ** 2, axis=-1, keepdims=True)
    return (y - mu) * jax.lax.rsqrt(var + 1e-5) * g + b


def _node_enc_body(x_ref, w1_ref, b1_ref, w2_ref, b2_ref, g_ref, be_ref, o_ref):
    h1 = jnp.maximum(
        jnp.dot(x_ref[...], w1_ref[...], preferred_element_type=jnp.float32)
        + b1_ref[...], 0.0)
    y = jnp.dot(h1, w2_ref[...], preferred_element_type=jnp.float32) + b2_ref[...]
    o_ref[...] = _ln_rows(y, g_ref[...], be_ref[...])


def _node_encoder(xp, p):
    rep = lambda *_: (0, 0)
    return pl.pallas_call(
        _node_enc_body,
        grid=(NB_N,),
        in_specs=[
            pl.BlockSpec((BN, F), lambda i: (i, 0)),
            pl.BlockSpec((F, D), rep),
            pl.BlockSpec((1, D), rep),
            pl.BlockSpec((D, D), rep),
            pl.BlockSpec((1, D), rep),
            pl.BlockSpec((1, D), rep),
            pl.BlockSpec((1, D), rep),
        ],
        out_specs=pl.BlockSpec((BN, D), lambda i: (i, 0)),
        out_shape=jax.ShapeDtypeStruct((NP, D), jnp.float32),
    )(xp, p['ne_W1'], p['ne_b1'][None], p['ne_W2'], p['ne_b2'][None],
      p['ne_g'][None], p['ne_be'][None])


def _edge_enc_body(ea_ref, w1_ref, b1_ref, w2_ref, b2_ref, g_ref, be_ref,
                   v_ref, a0_ref, a1_ref):
    h1 = jnp.maximum(
        jnp.dot(ea_ref[...], w1_ref[...], preferred_element_type=jnp.float32)
        + b1_ref[...], 0.0)
    y = jnp.dot(h1, w2_ref[...], preferred_element_type=jnp.float32) + b2_ref[...]
    ln = _ln_rows(y, g_ref[...], be_ref[...])
    a0_ref[...] = jnp.sum(ln * v_ref[0:1, :], axis=-1, keepdims=True)
    a1_ref[...] = jnp.sum(ln * v_ref[1:2, :], axis=-1, keepdims=True)


def _edge_encoder(edge_attr, p, v01):
    rep = lambda *_: (0, 0)
    a0, a1 = pl.pallas_call(
        _edge_enc_body,
        grid=(NB_E,),
        in_specs=[
            pl.BlockSpec((BE, FE), lambda i: (i, 0)),
            pl.BlockSpec((FE, D), rep),
            pl.BlockSpec((1, D), rep),
            pl.BlockSpec((D, D), rep),
            pl.BlockSpec((1, D), rep),
            pl.BlockSpec((1, D), rep),
            pl.BlockSpec((1, D), rep),
            pl.BlockSpec((2, D), rep),
        ],
        out_specs=[
            pl.BlockSpec((BE, 1), lambda i: (i, 0)),
            pl.BlockSpec((BE, 1), lambda i: (i, 0)),
        ],
        out_shape=[
            jax.ShapeDtypeStruct((E, 1), jnp.float32),
            jax.ShapeDtypeStruct((E, 1), jnp.float32),
        ],
    )(edge_attr, p['ee_W1'], p['ee_b1'][None], p['ee_W2'], p['ee_b2'][None],
      p['ee_g'][None], p['ee_be'][None], v01)
    return a0.reshape(E), a1.reshape(E)


def _proj_body(h_ref, w_ref, att_ref, xs_ref, as_ref, ad_ref):
    xs = jnp.dot(h_ref[...], w_ref[...], preferred_element_type=jnp.float32)
    xs_ref[...] = xs
    as_ref[...] = jnp.sum(xs * att_ref[0:1, :], axis=-1, keepdims=True)
    ad_ref[...] = jnp.sum(xs * att_ref[1:2, :], axis=-1, keepdims=True)


def _proj(h, W, att_sd):
    rep = lambda *_: (0, 0)
    xs, a_s, a_d = pl.pallas_call(
        _proj_body,
        grid=(NB_N,),
        in_specs=[
            pl.BlockSpec((BN, D), lambda i: (i, 0)),
            pl.BlockSpec((D, D), rep),
            pl.BlockSpec((2, D), rep),
        ],
        out_specs=[
            pl.BlockSpec((BN, D), lambda i: (i, 0)),
            pl.BlockSpec((BN, 1), lambda i: (i, 0)),
            pl.BlockSpec((BN, 1), lambda i: (i, 0)),
        ],
        out_shape=[
            jax.ShapeDtypeStruct((NP, D), jnp.float32),
            jax.ShapeDtypeStruct((NP, 1), jnp.float32),
            jax.ShapeDtypeStruct((NP, 1), jnp.float32),
        ],
    )(h, W, att_sd)
    return xs, a_s.reshape(NP), a_d.reshape(NP)


def _assemble_body(acc_ref, xs_ref, cl_ref, bias_ref, g_ref, be_ref, o_ref):
    y = acc_ref[...] + cl_ref[...] * xs_ref[...] + bias_ref[...]
    o_ref[...] = _ln_rows(y, g_ref[...], be_ref[...])


def _assemble(acc, xs, coef_loop, bias, g, be):
    rep = lambda *_: (0, 0)
    return pl.pallas_call(
        _assemble_body,
        grid=(NB_N,),
        in_specs=[
            pl.BlockSpec((BN, D), lambda i: (i, 0)),
            pl.BlockSpec((BN, D), lambda i: (i, 0)),
            pl.BlockSpec((BN, 1), lambda i: (i, 0)),
            pl.BlockSpec((1, D), rep),
            pl.BlockSpec((1, D), rep),
            pl.BlockSpec((1, D), rep),
        ],
        out_specs=pl.BlockSpec((BN, D), lambda i: (i, 0)),
        out_shape=jax.ShapeDtypeStruct((NP, D), jnp.float32),
    )(acc, xs, coef_loop.reshape(NP, 1), bias[None], g[None], be[None])


def _dec_body(h_ref, w1_ref, b1_ref, w2_ref, b2_ref, o_ref):
    h1 = jnp.maximum(
        jnp.dot(h_ref[...], w1_ref[...], preferred_element_type=jnp.float32)
        + b1_ref[...], 0.0)
    o_ref[...] = (jnp.dot(h1, w2_ref[...], preferred_element_type=jnp.float32)
                  + b2_ref[...])


def _decoder(h, p):
    rep = lambda *_: (0, 0)
    w2p = jnp.zeros((D, 128), jnp.float32).at[:, :OUT].set(p['dec_W2'])
    b2p = jnp.zeros((128,), jnp.float32).at[:OUT].set(p['dec_b2'])
    out = pl.pallas_call(
        _dec_body,
        grid=(NB_N,),
        in_specs=[
            pl.BlockSpec((BN, D), lambda i: (i, 0)),
            pl.BlockSpec((D, D), rep),
            pl.BlockSpec((1, D), rep),
            pl.BlockSpec((D, 128), rep),
            pl.BlockSpec((1, 128), rep),
        ],
        out_specs=pl.BlockSpec((BN, 128), lambda i: (i, 0)),
        out_shape=jax.ShapeDtypeStruct((NP, 128), jnp.float32),
    )(h, p['dec_W1'], p['dec_b1'][None], w2p, b2p[None])
    return out[:N, :OUT]


_MESH = plsc.VectorSubcoreMesh(core_axis_name="c", subcore_axis_name="s")
_CHA = 128
_NCHA = E // _CHA
_CHA_FULL = _NCHA // 32
_CHA_LEFT = _NCHA - 32 * _CHA_FULL
_CHC = 128
_NCHC = E // _CHC
_CHC_FULL = _NCHC // 32
_CHC_LEFT = _NCHC - 32 * _CHC_FULL

_i32 = jnp.int32
_f32 = jnp.float32


def _worker_id():
    return lax.axis_index("s") * 2 + lax.axis_index("c")


@functools.partial(
    pl.kernel, mesh=_MESH,
    out_type=jax.ShapeDtypeStruct((E,), _f32),
    scratch_types=[pltpu.VMEM((_CHA,), _i32), pltpu.VMEM((_CHA,), _i32),
                   pltpu.VMEM((_CHA,), _f32), pltpu.VMEM((_CHA,), _f32),
                   pltpu.VMEM((_CHA,), _f32), pltpu.VMEM((_CHA,), _f32),
                   pltpu.VMEM((_CHA,), _f32),
                   pltpu.SemaphoreType.DMA, pltpu.SemaphoreType.DMA])
def _sc_edge(src_hbm, dst_hbm, ae_hbm, as_hbm, ad_hbm, c_hbm, out_hbm,
             sbuf, dbuf, aebuf, asg, adg, cg, exbuf, sem, sem2):
    w = _worker_id()

    def chunk(cid):
        base = cid * _CHA
        pltpu.sync_copy(src_hbm.at[pl.ds(base, _CHA)], sbuf)
        pltpu.sync_copy(dst_hbm.at[pl.ds(base, _CHA)], dbuf)
        pltpu.sync_copy(ae_hbm.at[pl.ds(base, _CHA)], aebuf)
        g1 = pltpu.async_copy(as_hbm.at[sbuf], asg, sem2)
        g2 = pltpu.async_copy(ad_hbm.at[dbuf], adg, sem)
        g3 = pltpu.async_copy(c_hbm.at[dbuf], cg, sem)
        g1.wait()
        g2.wait()
        g3.wait()
        for g in range(_CHA // 16):
            sl = pl.ds(g * 16, 16)
            a = asg[sl] + adg[sl] + aebuf[sl]
            a = jnp.where(a > 0, a, 0.2 * a)
            exbuf[sl] = jnp.exp(jnp.minimum(a - cg[sl], 80.0))
        pltpu.sync_copy(exbuf, out_hbm.at[pl.ds(base, _CHA)])

    def body(i, _):
        chunk(w + i * 32)
        return 0

    lax.fori_loop(0, _CHA_FULL, body, 0)
    pl.when(w < _CHA_LEFT)(lambda: chunk(32 * _CHA_FULL + w))


@functools.partial(
    pl.kernel, mesh=_MESH,
    out_type=jax.ShapeDtypeStruct((E, D), _f32),
    scratch_types=[pltpu.VMEM((_CHC,), _i32), pltpu.VMEM((_CHC,), _i32),
                   pltpu.VMEM((_CHC,), _f32), pltpu.VMEM((_CHC,), _f32),
                   pltpu.VMEM((_CHC,), _f32), pltpu.VMEM((_CHC, D), _f32),
                   pltpu.SemaphoreType.DMA, pltpu.SemaphoreType.DMA])
def _sc_msg(src_hbm, dst_hbm, ex_hbm, inv_hbm, xs_hbm, out_hbm,
            sbuf, dbuf, exbuf, invg, coefb, rows, sem, sem2):
    w = _worker_id()

    def chunk(cid):
        base = cid * _CHC
        pltpu.sync_copy(src_hbm.at[pl.ds(base, _CHC)], sbuf)
        pltpu.sync_copy(dst_hbm.at[pl.ds(base, _CHC)], dbuf)
        pltpu.sync_copy(ex_hbm.at[pl.ds(base, _CHC)], exbuf)
        gat = pltpu.async_copy(xs_hbm.at[sbuf], rows, sem2)
        pltpu.async_copy(inv_hbm.at[dbuf], invg, sem).wait()
        for g in range(_CHC // 16):
            sl = pl.ds(g * 16, 16)
            coefb[sl] = exbuf[sl] * invg[sl]
        gat.wait()
        for g in range(_CHC // 16):
            cg = coefb[pl.ds(g * 16, 16)]
            for r in range(16):
                row = g * 16 + r
                cs = cg[r]
                for q in range(D // 16):
                    qs = pl.ds(q * 16, 16)
                    rows[row, qs] = rows[row, qs] * cs
        pltpu.sync_copy(rows, out_hbm.at[pl.ds(base, _CHC)])

    def body(i, _):
        chunk(w + i * 32)
        return 0

    lax.fori_loop(0, _CHC_FULL, body, 0)
    pl.when(w < _CHC_LEFT)(lambda: chunk(32 * _CHC_FULL + w))


def kernel(x, edge_attr, edge_index, batch, params):
    p = params
    src, dst = edge_index[0], edge_index[1]

    xp = jnp.zeros((NP, F), jnp.float32).at[:N].set(x)
    h = _node_encoder(xp, p)

    v01 = jnp.stack([p['gat0_We'] @ p['gat0_att_e'],
                     p['gat1_We'] @ p['gat1_att_e']], axis=0)
    ae0, ae1 = _edge_encoder(edge_attr, p, v01)
    ae_by_layer = (ae0, ae1)

    for l in range(2):
        att_sd = jnp.stack([p['gat%d_att_s' % l], p['gat%d_att_d' % l]], axis=0)
        xs, a_s, a_d = _proj(h, p['gat%d_W' % l], att_sd)
        ae = ae_by_layer[l]
        ae_loop = jnp.mean(ae)

        alpha_loop = a_s + a_d + ae_loop
        alpha_loop = jnp.where(alpha_loop > 0, alpha_loop, 0.2 * alpha_loop)
        ex = _sc_edge(src, dst, ae, a_s, a_d, alpha_loop)
        denom = jax.ops.segment_sum(ex, dst, num_segments=NP) + 1.0
        inv = 1.0 / (denom + 1e-16)
        msg = _sc_msg(src, dst, ex, inv, xs)
        acc = jax.ops.segment_sum(msg, dst, num_segments=NP)

        h = _assemble(acc, xs, inv, p['gat%d_bias' % l],
                      p['gat%d_g' % l], p['gat%d_be' % l])

    return _decoder(h, p)


# speedup vs baseline: 5.0012x; 1.0116x over previous
import functools

import jax
import jax.numpy as jnp
from jax import lax
from jax.experimental import pallas as pl
from jax.experimental.pallas import tpu as pltpu
from jax.experimental.pallas import tpu_sc as plsc

N = 10000
E = 320000
F = 128
FE = 16
D = 256
OUT = 4

NP = 10240
NB_N = 10
BN = NP // NB_N
BE = 3200
NB_E = E // BE


def _ln_rows(y, g, b):
    mu = jnp.mean(y, axis=-1, keepdims=True)
    var = jnp.mean((y - mu) ** 2, axis=-1, keepdims=True)
    return (y - mu) * jax.lax.rsqrt(var + 1e-5) * g + b


def _node_enc_body(x_ref, w1_ref, b1_ref, w2_ref, b2_ref, g_ref, be_ref, o_ref):
    h1 = jnp.maximum(
        jnp.dot(x_ref[...], w1_ref[...], preferred_element_type=jnp.float32)
        + b1_ref[...], 0.0)
    y = jnp.dot(h1, w2_ref[...], preferred_element_type=jnp.float32) + b2_ref[...]
    o_ref[...] = _ln_rows(y, g_ref[...], be_ref[...])


def _node_encoder(xp, p):
    rep = lambda *_: (0, 0)
    return pl.pallas_call(
        _node_enc_body,
        grid=(NB_N,),
        in_specs=[
            pl.BlockSpec((BN, F), lambda i: (i, 0)),
            pl.BlockSpec((F, D), rep),
            pl.BlockSpec((1, D), rep),
            pl.BlockSpec((D, D), rep),
            pl.BlockSpec((1, D), rep),
            pl.BlockSpec((1, D), rep),
            pl.BlockSpec((1, D), rep),
        ],
        out_specs=pl.BlockSpec((BN, D), lambda i: (i, 0)),
        out_shape=jax.ShapeDtypeStruct((NP, D), jnp.float32),
    )(xp, p['ne_W1'], p['ne_b1'][None], p['ne_W2'], p['ne_b2'][None],
      p['ne_g'][None], p['ne_be'][None])


def _edge_enc_body(ea_ref, w1_ref, b1_ref, w2_ref, b2_ref, g_ref, be_ref,
                   v_ref, a0_ref, a1_ref):
    h1 = jnp.maximum(
        jnp.dot(ea_ref[...], w1_ref[...], preferred_element_type=jnp.float32)
        + b1_ref[...], 0.0)
    y = jnp.dot(h1, w2_ref[...], preferred_element_type=jnp.float32) + b2_ref[...]
    ln = _ln_rows(y, g_ref[...], be_ref[...])
    a0_ref[...] = jnp.sum(ln * v_ref[0:1, :], axis=-1, keepdims=True)
    a1_ref[...] = jnp.sum(ln * v_ref[1:2, :], axis=-1, keepdims=True)


def _edge_encoder(edge_attr, p, v01):
    rep = lambda *_: (0, 0)
    a0, a1 = pl.pallas_call(
        _edge_enc_body,
        grid=(NB_E,),
        in_specs=[
            pl.BlockSpec((BE, FE), lambda i: (i, 0)),
            pl.BlockSpec((FE, D), rep),
            pl.BlockSpec((1, D), rep),
            pl.BlockSpec((D, D), rep),
            pl.BlockSpec((1, D), rep),
            pl.BlockSpec((1, D), rep),
            pl.BlockSpec((1, D), rep),
            pl.BlockSpec((2, D), rep),
        ],
        out_specs=[
            pl.BlockSpec((BE, 1), lambda i: (i, 0)),
            pl.BlockSpec((BE, 1), lambda i: (i, 0)),
        ],
        out_shape=[
            jax.ShapeDtypeStruct((E, 1), jnp.float32),
            jax.ShapeDtypeStruct((E, 1), jnp.float32),
        ],
    )(edge_attr, p['ee_W1'], p['ee_b1'][None], p['ee_W2'], p['ee_b2'][None],
      p['ee_g'][None], p['ee_be'][None], v01)
    return a0.reshape(E), a1.reshape(E)


def _proj_body(h_ref, w_ref, att_ref, xs_ref, as_ref, ad_ref):
    xs = jnp.dot(h_ref[...], w_ref[...], preferred_element_type=jnp.float32)
    xs_ref[...] = xs
    as_ref[...] = jnp.sum(xs * att_ref[0:1, :], axis=-1, keepdims=True)
    ad_ref[...] = jnp.sum(xs * att_ref[1:2, :], axis=-1, keepdims=True)


def _proj(h, W, att_sd):
    rep = lambda *_: (0, 0)
    xs, a_s, a_d = pl.pallas_call(
        _proj_body,
        grid=(NB_N,),
        in_specs=[
            pl.BlockSpec((BN, D), lambda i: (i, 0)),
            pl.BlockSpec((D, D), rep),
            pl.BlockSpec((2, D), rep),
        ],
        out_specs=[
            pl.BlockSpec((BN, D), lambda i: (i, 0)),
            pl.BlockSpec((BN, 1), lambda i: (i, 0)),
            pl.BlockSpec((BN, 1), lambda i: (i, 0)),
        ],
        out_shape=[
            jax.ShapeDtypeStruct((NP, D), jnp.float32),
            jax.ShapeDtypeStruct((NP, 1), jnp.float32),
            jax.ShapeDtypeStruct((NP, 1), jnp.float32),
        ],
    )(h, W, att_sd)
    return xs, a_s.reshape(NP), a_d.reshape(NP)


def _assemble_body(acc_ref, xs_ref, cl_ref, bias_ref, g_ref, be_ref, o_ref):
    y = acc_ref[...] + cl_ref[...] * xs_ref[...] + bias_ref[...]
    o_ref[...] = _ln_rows(y, g_ref[...], be_ref[...])


def _assemble(acc, xs, coef_loop, bias, g, be):
    rep = lambda *_: (0, 0)
    return pl.pallas_call(
        _assemble_body,
        grid=(NB_N,),
        in_specs=[
            pl.BlockSpec((BN, D), lambda i: (i, 0)),
            pl.BlockSpec((BN, D), lambda i: (i, 0)),
            pl.BlockSpec((BN, 1), lambda i: (i, 0)),
            pl.BlockSpec((1, D), rep),
            pl.BlockSpec((1, D), rep),
            pl.BlockSpec((1, D), rep),
        ],
        out_specs=pl.BlockSpec((BN, D), lambda i: (i, 0)),
        out_shape=jax.ShapeDtypeStruct((NP, D), jnp.float32),
    )(acc, xs, coef_loop.reshape(NP, 1), bias[None], g[None], be[None])


def _dec_body(h_ref, w1_ref, b1_ref, w2_ref, b2_ref, o_ref):
    h1 = jnp.maximum(
        jnp.dot(h_ref[...], w1_ref[...], preferred_element_type=jnp.float32)
        + b1_ref[...], 0.0)
    o_ref[...] = (jnp.dot(h1, w2_ref[...], preferred_element_type=jnp.float32)
                  + b2_ref[...])


def _decoder(h, p):
    rep = lambda *_: (0, 0)
    w2p = jnp.zeros((D, 128), jnp.float32).at[:, :OUT].set(p['dec_W2'])
    b2p = jnp.zeros((128,), jnp.float32).at[:OUT].set(p['dec_b2'])
    out = pl.pallas_call(
        _dec_body,
        grid=(NB_N,),
        in_specs=[
            pl.BlockSpec((BN, D), lambda i: (i, 0)),
            pl.BlockSpec((D, D), rep),
            pl.BlockSpec((1, D), rep),
            pl.BlockSpec((D, 128), rep),
            pl.BlockSpec((1, 128), rep),
        ],
        out_specs=pl.BlockSpec((BN, 128), lambda i: (i, 0)),
        out_shape=jax.ShapeDtypeStruct((NP, 128), jnp.float32),
    )(h, p['dec_W1'], p['dec_b1'][None], w2p, b2p[None])
    return out[:N, :OUT]


_MESH = plsc.VectorSubcoreMesh(core_axis_name="c", subcore_axis_name="s")
_CHA = 128
_NCHA = E // _CHA
_CHA_FULL = _NCHA // 32
_CHA_LEFT = _NCHA - 32 * _CHA_FULL
_CHC = 128
_NCHC = E // _CHC
_CHC_FULL = _NCHC // 32
_CHC_LEFT = _NCHC - 32 * _CHC_FULL

_i32 = jnp.int32
_f32 = jnp.float32


def _worker_id():
    return lax.axis_index("s") * 2 + lax.axis_index("c")


@functools.partial(
    pl.kernel, mesh=_MESH,
    out_type=jax.ShapeDtypeStruct((E,), _f32),
    scratch_types=[pltpu.VMEM((_CHA,), _i32), pltpu.VMEM((_CHA,), _i32),
                   pltpu.VMEM((_CHA,), _f32), pltpu.VMEM((_CHA,), _f32),
                   pltpu.VMEM((_CHA,), _f32), pltpu.VMEM((_CHA,), _f32),
                   pltpu.VMEM((_CHA,), _f32),
                   pltpu.SemaphoreType.DMA, pltpu.SemaphoreType.DMA])
def _sc_edge(src_hbm, dst_hbm, ae_hbm, as_hbm, ad_hbm, c_hbm, out_hbm,
             sbuf, dbuf, aebuf, asg, adg, cg, exbuf, sem, sem2):
    w = _worker_id()

    def chunk(cid):
        base = cid * _CHA
        pltpu.sync_copy(src_hbm.at[pl.ds(base, _CHA)], sbuf)
        pltpu.sync_copy(dst_hbm.at[pl.ds(base, _CHA)], dbuf)
        pltpu.sync_copy(ae_hbm.at[pl.ds(base, _CHA)], aebuf)
        g1 = pltpu.async_copy(as_hbm.at[sbuf], asg, sem2)
        g2 = pltpu.async_copy(ad_hbm.at[dbuf], adg, sem)
        g3 = pltpu.async_copy(c_hbm.at[dbuf], cg, sem)
        g1.wait()
        g2.wait()
        g3.wait()
        for g in range(_CHA // 16):
            sl = pl.ds(g * 16, 16)
            a = asg[sl] + adg[sl] + aebuf[sl]
            a = jnp.where(a > 0, a, 0.2 * a)
            exbuf[sl] = jnp.exp(jnp.minimum(a - cg[sl], 80.0))
        pltpu.sync_copy(exbuf, out_hbm.at[pl.ds(base, _CHA)])

    def body(i, _):
        chunk(w + i * 32)
        return 0

    lax.fori_loop(0, _CHA_FULL, body, 0)
    pl.when(w < _CHA_LEFT)(lambda: chunk(32 * _CHA_FULL + w))


@functools.partial(
    pl.kernel, mesh=_MESH,
    out_type=jax.ShapeDtypeStruct((E, D), _f32),
    scratch_types=[pltpu.VMEM((_CHC,), _i32), pltpu.VMEM((_CHC,), _i32),
                   pltpu.VMEM((_CHC,), _f32), pltpu.VMEM((_CHC,), _f32),
                   pltpu.VMEM((_CHC,), _f32), pltpu.VMEM((_CHC, D), _f32),
                   pltpu.SemaphoreType.DMA, pltpu.SemaphoreType.DMA])
def _sc_msg(src_hbm, dst_hbm, ex_hbm, inv_hbm, xs_hbm, out_hbm,
            sbuf, dbuf, exbuf, invg, coefb, rows, sem, sem2):
    w = _worker_id()

    def chunk(cid):
        base = cid * _CHC
        c1 = pltpu.async_copy(src_hbm.at[pl.ds(base, _CHC)], sbuf, sem)
        c2 = pltpu.async_copy(dst_hbm.at[pl.ds(base, _CHC)], dbuf, sem)
        c3 = pltpu.async_copy(ex_hbm.at[pl.ds(base, _CHC)], exbuf, sem)
        c1.wait()
        c2.wait()
        c3.wait()
        gat = pltpu.async_copy(xs_hbm.at[sbuf], rows, sem2)
        pltpu.async_copy(inv_hbm.at[dbuf], invg, sem).wait()
        for g in range(_CHC // 16):
            sl = pl.ds(g * 16, 16)
            coefb[sl] = exbuf[sl] * invg[sl]
        gat.wait()
        for g in range(_CHC // 16):
            cg = coefb[pl.ds(g * 16, 16)]
            for r in range(16):
                row = g * 16 + r
                cs = cg[r]
                for q in range(D // 16):
                    qs = pl.ds(q * 16, 16)
                    rows[row, qs] = rows[row, qs] * cs
        pltpu.sync_copy(rows, out_hbm.at[pl.ds(base, _CHC)])

    def body(i, _):
        chunk(w + i * 32)
        return 0

    lax.fori_loop(0, _CHC_FULL, body, 0)
    pl.when(w < _CHC_LEFT)(lambda: chunk(32 * _CHC_FULL + w))


def kernel(x, edge_attr, edge_index, batch, params):
    p = params
    src, dst = edge_index[0], edge_index[1]

    xp = jnp.zeros((NP, F), jnp.float32).at[:N].set(x)
    h = _node_encoder(xp, p)

    v01 = jnp.stack([p['gat0_We'] @ p['gat0_att_e'],
                     p['gat1_We'] @ p['gat1_att_e']], axis=0)
    ae0, ae1 = _edge_encoder(edge_attr, p, v01)
    ae_by_layer = (ae0, ae1)

    for l in range(2):
        att_sd = jnp.stack([p['gat%d_att_s' % l], p['gat%d_att_d' % l]], axis=0)
        xs, a_s, a_d = _proj(h, p['gat%d_W' % l], att_sd)
        ae = ae_by_layer[l]
        ae_loop = jnp.mean(ae)

        alpha_loop = a_s + a_d + ae_loop
        alpha_loop = jnp.where(alpha_loop > 0, alpha_loop, 0.2 * alpha_loop)
        ex = _sc_edge(src, dst, ae, a_s, a_d, alpha_loop)
        denom = jax.ops.segment_sum(ex, dst, num_segments=NP) + 1.0
        inv = 1.0 / (denom + 1e-16)
        msg = _sc_msg(src, dst, ex, inv, xs)
        acc = jax.ops.segment_sum(msg, dst, num_segments=NP)

        h = _assemble(acc, xs, inv, p['gat%d_bias' % l],
                      p['gat%d_g' % l], p['gat%d_be' % l])

    return _decoder(h, p)
